# baseline (speedup 1.0000x reference)
import functools

import jax
import jax.numpy as jnp
from jax import lax
from jax.experimental import pallas as pl
from jax.experimental.pallas import tpu as pltpu
from jax.experimental.pallas import tpu_sc as plsc

DT = 0.03

_NC = 2
_NS = 16
_NW = _NC * _NS


def _gather_body(h_hbm, row_hbm, col_hbm, hi_hbm, hj_hbm,
                 ri0, ri1, ci0, ci1, ra0, ra1, ca0, ca1, sem, wsem,
                 *, epw, ch):
    c = lax.axis_index("c")
    s = lax.axis_index("s")
    wid = s * _NC + c
    base = wid * epw

    def body(k, _):
        off0 = base + (2 * k) * ch
        off1 = off0 + ch
        pltpu.sync_copy(row_hbm.at[pl.ds(off0, ch)], ri0)
        pltpu.sync_copy(row_hbm.at[pl.ds(off1, ch)], ri1)
        pltpu.sync_copy(col_hbm.at[pl.ds(off0, ch)], ci0)
        pltpu.sync_copy(col_hbm.at[pl.ds(off1, ch)], ci1)
        d0 = pltpu.async_copy(h_hbm.at[ri0], ra0, sem)
        d1 = pltpu.async_copy(h_hbm.at[ri1], ra1, sem)
        d2 = pltpu.async_copy(h_hbm.at[ci0], ca0, sem)
        d3 = pltpu.async_copy(h_hbm.at[ci1], ca1, sem)
        d0.wait(); d1.wait(); d2.wait(); d3.wait()
        w0 = pltpu.async_copy(ra0, hi_hbm.at[pl.ds(off0, ch)], wsem)
        w1 = pltpu.async_copy(ra1, hi_hbm.at[pl.ds(off1, ch)], wsem)
        w2 = pltpu.async_copy(ca0, hj_hbm.at[pl.ds(off0, ch)], wsem)
        w3 = pltpu.async_copy(ca1, hj_hbm.at[pl.ds(off1, ch)], wsem)
        w0.wait(); w1.wait(); w2.wait(); w3.wait()
        return 0

    lax.fori_loop(0, epw // (2 * ch), body, 0)


def _sc_gather(h_pack, row, col):
    e = row.shape[0]
    dw = h_pack.shape[1]
    epw = e // _NW
    ch = 200
    mesh = plsc.VectorSubcoreMesh(core_axis_name="c", subcore_axis_name="s")
    kern = pl.kernel(
        functools.partial(_gather_body, epw=epw, ch=ch),
        compiler_params=pltpu.CompilerParams(use_tc_tiling_on_sc=False),
        out_type=(
            jax.ShapeDtypeStruct((e, dw), h_pack.dtype),
            jax.ShapeDtypeStruct((e, dw), h_pack.dtype),
        ),
        mesh=mesh,
        scratch_types=[
            pltpu.VMEM((ch,), jnp.int32),
            pltpu.VMEM((ch,), jnp.int32),
            pltpu.VMEM((ch,), jnp.int32),
            pltpu.VMEM((ch,), jnp.int32),
            pltpu.VMEM((ch, dw), h_pack.dtype),
            pltpu.VMEM((ch, dw), h_pack.dtype),
            pltpu.VMEM((ch, dw), h_pack.dtype),
            pltpu.VMEM((ch, dw), h_pack.dtype),
            pltpu.SemaphoreType.DMA,
            pltpu.SemaphoreType.DMA,
        ],
    )
    return kern(h_pack, row, col)


def _scatter_body(msg_hbm, row_hbm, zeros_hbm, out_hbm,
                  idx_v, msg_v, shared, sem, *, epw, ch, nps):
    c = lax.axis_index("c")
    s = lax.axis_index("s")
    wid = s * _NC + c
    base = wid * epw

    pltpu.sync_copy(zeros_hbm, shared.at[pl.ds(s * nps, nps)])
    plsc.subcore_barrier()

    def body(i, _):
        off = base + i * ch
        d0 = pltpu.async_copy(row_hbm.at[pl.ds(off, ch)], idx_v, sem)
        d1 = pltpu.async_copy(msg_hbm.at[pl.ds(off, ch)], msg_v, sem)
        d0.wait(); d1.wait()
        pltpu.sync_copy(msg_v, shared.at[idx_v], add=True)
        return 0

    lax.fori_loop(0, epw // ch, body, 0)
    plsc.subcore_barrier()

    npad = nps * _NS
    pltpu.sync_copy(shared.at[pl.ds(s * nps, nps)],
                    out_hbm.at[pl.ds(c * npad + s * nps, nps)])


def _sc_scatter(msg, row, n_pad):
    e, d = msg.shape
    epw = e // _NW
    ch = 200
    nps = n_pad // _NS
    zeros = jnp.zeros((nps, d), msg.dtype)
    mesh = plsc.VectorSubcoreMesh(core_axis_name="c", subcore_axis_name="s")
    kern = pl.kernel(
        functools.partial(_scatter_body, epw=epw, ch=ch, nps=nps),
        out_type=jax.ShapeDtypeStruct((_NC * n_pad, d), msg.dtype),
        mesh=mesh,
        scratch_types=[
            pltpu.VMEM((ch,), jnp.int32),
            pltpu.VMEM((ch, d), msg.dtype),
            pltpu.VMEM_SHARED((n_pad, d), msg.dtype),
            pltpu.SemaphoreType.DMA,
        ],
    )
    return kern(msg, row, zeros)


def _mlp_body(hi_ref, hj_ref, vw1a, vw1b, vb1, vw2r, vb2,
              pw1, pb1, pw2, pb2, fw1a, fw1b, fb1, fw2, fb2, out_ref):
    f32 = jnp.float32
    bf = jnp.bfloat16
    hi = hi_ref[...]
    hj = hj_ref[...]
    hd = (hi.astype(f32) - hj.astype(f32))
    tv = jnp.tanh(jnp.dot(hi, vw1a[...], preferred_element_type=f32)
                  + jnp.dot(hj, vw1b[...], preferred_element_type=f32)
                  + vb1[...])
    nu = jnp.sum(tv * vw2r[...], axis=1, keepdims=True) + vb2[...]
    diff = nu * (-hd)
    tf = jax.nn.relu(jnp.dot(hi, fw1a[...], preferred_element_type=f32)
                     + jnp.dot(hj, fw1b[...], preferred_element_type=f32)
                     + fb1[...])
    force = jnp.dot(tf.astype(bf), fw2[...], preferred_element_type=f32) + fb2[...]
    tp = jnp.tanh(jnp.dot(hd.astype(bf), pw1[...], preferred_element_type=f32)
                  + pb1[...])
    pres = jnp.dot(tp.astype(bf), pw2[...], preferred_element_type=f32) + pb2[...]
    out_ref[...] = diff + force - pres


def _tc_mlp(hi, hj, weights):
    e, d = hi.shape
    be = 2000
    grid = e // be
    row_spec = pl.BlockSpec((be, d), lambda i: (i, 0))
    full = lambda a: pl.BlockSpec(a.shape, lambda i: tuple(0 for _ in a.shape))
    return pl.pallas_call(
        _mlp_body,
        out_shape=jax.ShapeDtypeStruct((e, d), jnp.float32),
        grid=(grid,),
        in_specs=[row_spec, row_spec] + [full(w) for w in weights],
        out_specs=row_spec,
    )(hi, hj, *weights)


def _combine_body(h_ref, p0_ref, p1_ref, out_ref):
    out_ref[...] = h_ref[...] + DT * (p0_ref[...] + p1_ref[...])


def _tc_combine(h, partials, n_pad):
    n, d = h.shape
    bn = 80
    spec = pl.BlockSpec((bn, d), lambda i: (i, 0))
    p1_spec = pl.BlockSpec((bn, d), lambda i: (i + n_pad // bn, 0))
    return pl.pallas_call(
        _combine_body,
        out_shape=jax.ShapeDtypeStruct((n, d), jnp.float32),
        grid=(n // bn,),
        in_specs=[spec, spec, p1_spec],
        out_specs=spec,
    )(h, partials, partials)


def kernel(h, edge_index, vw1, vb1, vw2, vb2, pw1, pb1, pw2, pb2,
           fw1, fb1, fw2, fb2):
    n, d = h.shape
    f32 = jnp.float32
    bf = jnp.bfloat16
    row = edge_index[0]
    col = edge_index[1]

    h_bf = h.astype(bf)
    h_pack = lax.bitcast_convert_type(h_bf.reshape(n, d // 2, 2), jnp.int32)
    hi_pack, hj_pack = _sc_gather(h_pack, row, col)
    e = row.shape[0]
    hi = lax.bitcast_convert_type(hi_pack, bf).reshape(e, d)
    hj = lax.bitcast_convert_type(hj_pack, bf).reshape(e, d)

    weights = (
        vw1[:d].astype(bf), vw1[d:].astype(bf), vb1.reshape(1, d).astype(f32),
        vw2.reshape(1, d).astype(f32), vb2.reshape(1, 1).astype(f32),
        pw1.astype(bf), pb1.reshape(1, d).astype(f32),
        pw2.astype(bf), pb2.reshape(1, d).astype(f32),
        fw1[:d].astype(bf), fw1[d:].astype(bf), fb1.reshape(1, d).astype(f32),
        fw2.astype(bf), fb2.reshape(1, d).astype(f32),
    )
    msg = _tc_mlp(hi, hj, weights)

    n_pad = ((n + _NW * 8 - 1) // (_NW * 8)) * (_NW * 8)
    partials = _sc_scatter(msg, row, n_pad)

    return _tc_combine(h, partials, n_pad)


# speedup vs baseline: 2.3615x; 2.3615x over previous
import functools

import jax
import jax.numpy as jnp
from jax import lax
from jax.experimental import pallas as pl
from jax.experimental.pallas import tpu as pltpu
from jax.experimental.pallas import tpu_sc as plsc

DT = 0.03

_NC = 2
_NS = 16
_NW = _NC * _NS


def _gather_body(h_hbm, row_hbm, col_hbm, hi_hbm, hj_hbm,
                 ri0, ri1, ci0, ci1, ra0, ra1, ca0, ca1, sem, wsem,
                 *, epw, ch):
    c = lax.axis_index("c")
    s = lax.axis_index("s")
    wid = s * _NC + c
    base = wid * epw

    def body(k, _):
        off0 = base + (2 * k) * ch
        off1 = off0 + ch
        pltpu.sync_copy(row_hbm.at[pl.ds(off0, ch)], ri0)
        pltpu.sync_copy(row_hbm.at[pl.ds(off1, ch)], ri1)
        pltpu.sync_copy(col_hbm.at[pl.ds(off0, ch)], ci0)
        pltpu.sync_copy(col_hbm.at[pl.ds(off1, ch)], ci1)
        d0 = pltpu.async_copy(h_hbm.at[ri0], ra0, sem)
        d1 = pltpu.async_copy(h_hbm.at[ri1], ra1, sem)
        d2 = pltpu.async_copy(h_hbm.at[ci0], ca0, sem)
        d3 = pltpu.async_copy(h_hbm.at[ci1], ca1, sem)
        d0.wait(); d1.wait(); d2.wait(); d3.wait()
        w0 = pltpu.async_copy(ra0, hi_hbm.at[pl.ds(off0, ch)], wsem)
        w1 = pltpu.async_copy(ra1, hi_hbm.at[pl.ds(off1, ch)], wsem)
        w2 = pltpu.async_copy(ca0, hj_hbm.at[pl.ds(off0, ch)], wsem)
        w3 = pltpu.async_copy(ca1, hj_hbm.at[pl.ds(off1, ch)], wsem)
        w0.wait(); w1.wait(); w2.wait(); w3.wait()
        return 0

    lax.fori_loop(0, epw // (2 * ch), body, 0)


def _sc_gather(h_pack, row, col):
    e = row.shape[0]
    dw = h_pack.shape[1]
    epw = e // _NW
    ch = 200
    mesh = plsc.VectorSubcoreMesh(core_axis_name="c", subcore_axis_name="s")
    kern = pl.kernel(
        functools.partial(_gather_body, epw=epw, ch=ch),
        out_type=(
            jax.ShapeDtypeStruct((e, dw), h_pack.dtype),
            jax.ShapeDtypeStruct((e, dw), h_pack.dtype),
        ),
        mesh=mesh,
        scratch_types=[
            pltpu.VMEM((ch,), jnp.int32),
            pltpu.VMEM((ch,), jnp.int32),
            pltpu.VMEM((ch,), jnp.int32),
            pltpu.VMEM((ch,), jnp.int32),
            pltpu.VMEM((ch, dw), h_pack.dtype),
            pltpu.VMEM((ch, dw), h_pack.dtype),
            pltpu.VMEM((ch, dw), h_pack.dtype),
            pltpu.VMEM((ch, dw), h_pack.dtype),
            pltpu.SemaphoreType.DMA,
            pltpu.SemaphoreType.DMA,
        ],
    )
    return kern(h_pack, row, col)


def _scatter_body(msg_hbm, row_hbm, zeros_hbm, out_hbm,
                  idx_v, msg_v, shared, sem, *, epw, ch, nps):
    c = lax.axis_index("c")
    s = lax.axis_index("s")
    wid = s * _NC + c
    base = wid * epw

    pltpu.sync_copy(zeros_hbm, shared.at[pl.ds(s * nps, nps)])
    plsc.subcore_barrier()

    def body(i, _):
        off = base + i * ch
        d0 = pltpu.async_copy(row_hbm.at[pl.ds(off, ch)], idx_v, sem)
        d1 = pltpu.async_copy(msg_hbm.at[pl.ds(off, ch)], msg_v, sem)
        d0.wait(); d1.wait()
        pltpu.sync_copy(msg_v, shared.at[idx_v], add=True)
        return 0

    lax.fori_loop(0, epw // ch, body, 0)
    plsc.subcore_barrier()

    npad = nps * _NS
    pltpu.sync_copy(shared.at[pl.ds(s * nps, nps)],
                    out_hbm.at[pl.ds(c * npad + s * nps, nps)])


def _sc_scatter(msg, row, n_pad):
    e, d = msg.shape
    epw = e // _NW
    ch = 200
    nps = n_pad // _NS
    zeros = jnp.zeros((nps, d), msg.dtype)
    mesh = plsc.VectorSubcoreMesh(core_axis_name="c", subcore_axis_name="s")
    kern = pl.kernel(
        functools.partial(_scatter_body, epw=epw, ch=ch, nps=nps),
        out_type=jax.ShapeDtypeStruct((_NC * n_pad, d), msg.dtype),
        mesh=mesh,
        scratch_types=[
            pltpu.VMEM((ch,), jnp.int32),
            pltpu.VMEM((ch, d), msg.dtype),
            pltpu.VMEM_SHARED((n_pad, d), msg.dtype),
            pltpu.SemaphoreType.DMA,
        ],
    )
    return kern(msg, row, zeros)


def _mlp_body(hi_ref, hj_ref, vw1a, vw1b, vb1, vw2r, vb2,
              pw1, pb1, pw2, pb2, fw1a, fw1b, fb1, fw2, fb2, out_ref):
    f32 = jnp.float32
    bf = jnp.bfloat16
    hi = hi_ref[...]
    hj = hj_ref[...]
    hd = hi - hj
    hi_b = hi.astype(bf)
    hj_b = hj.astype(bf)
    tv = jnp.tanh(jnp.dot(hi_b, vw1a[...], preferred_element_type=f32)
                  + jnp.dot(hj_b, vw1b[...], preferred_element_type=f32)
                  + vb1[...])
    nu = jnp.sum(tv * vw2r[...], axis=1, keepdims=True) + vb2[...]
    diff = nu * (-hd)
    tf = jax.nn.relu(jnp.dot(hi_b, fw1a[...], preferred_element_type=f32)
                     + jnp.dot(hj_b, fw1b[...], preferred_element_type=f32)
                     + fb1[...])
    force = jnp.dot(tf.astype(bf), fw2[...], preferred_element_type=f32) + fb2[...]
    tp = jnp.tanh(jnp.dot(hd.astype(bf), pw1[...], preferred_element_type=f32)
                  + pb1[...])
    pres = jnp.dot(tp.astype(bf), pw2[...], preferred_element_type=f32) + pb2[...]
    out_ref[...] = diff + force - pres


def _tc_mlp(hi, hj, weights):
    e, d = hi.shape
    be = 2000
    grid = e // be
    row_spec = pl.BlockSpec((be, d), lambda i: (i, 0))
    full = lambda a: pl.BlockSpec(a.shape, lambda i: tuple(0 for _ in a.shape))
    return pl.pallas_call(
        _mlp_body,
        out_shape=jax.ShapeDtypeStruct((e, d), jnp.float32),
        grid=(grid,),
        in_specs=[row_spec, row_spec] + [full(w) for w in weights],
        out_specs=row_spec,
    )(hi, hj, *weights)


def _combine_body(h_ref, p0_ref, p1_ref, out_ref):
    out_ref[...] = h_ref[...] + DT * (p0_ref[...] + p1_ref[...])


def _tc_combine(h, partials, n_pad):
    n, d = h.shape
    bn = 80
    spec = pl.BlockSpec((bn, d), lambda i: (i, 0))
    p1_spec = pl.BlockSpec((bn, d), lambda i: (i + n_pad // bn, 0))
    return pl.pallas_call(
        _combine_body,
        out_shape=jax.ShapeDtypeStruct((n, d), jnp.float32),
        grid=(n // bn,),
        in_specs=[spec, spec, p1_spec],
        out_specs=spec,
    )(h, partials, partials)


def kernel(h, edge_index, vw1, vb1, vw2, vb2, pw1, pb1, pw2, pb2,
           fw1, fb1, fw2, fb2):
    n, d = h.shape
    f32 = jnp.float32
    bf = jnp.bfloat16
    row = edge_index[0]
    col = edge_index[1]

    hi, hj = _sc_gather(h, row, col)

    weights = (
        vw1[:d].astype(bf), vw1[d:].astype(bf), vb1.reshape(1, d).astype(f32),
        vw2.reshape(1, d).astype(f32), vb2.reshape(1, 1).astype(f32),
        pw1.astype(bf), pb1.reshape(1, d).astype(f32),
        pw2.astype(bf), pb2.reshape(1, d).astype(f32),
        fw1[:d].astype(bf), fw1[d:].astype(bf), fb1.reshape(1, d).astype(f32),
        fw2.astype(bf), fb2.reshape(1, d).astype(f32),
    )
    msg = _tc_mlp(hi, hj, weights)

    n_pad = ((n + _NW * 8 - 1) // (_NW * 8)) * (_NW * 8)
    partials = _sc_scatter(msg, row, n_pad)

    return _tc_combine(h, partials, n_pad)


# speedup vs baseline: 3.1878x; 1.3499x over previous
import functools

import jax
import jax.numpy as jnp
from jax import lax
from jax.experimental import pallas as pl
from jax.experimental.pallas import tpu as pltpu
from jax.experimental.pallas import tpu_sc as plsc

DT = 0.03

_NC = 2
_NS = 16
_NW = _NC * _NS


def _gather_body(h_hbm, row_hbm, col_hbm, hcat_hbm,
                 ri0, ri1, ci0, ci1, ra0, ra1, ca0, ca1, sem, wsem,
                 *, epw, ch, d):
    c = lax.axis_index("c")
    s = lax.axis_index("s")
    wid = s * _NC + c
    base = wid * epw

    def body(k, _):
        off0 = base + (2 * k) * ch
        off1 = off0 + ch
        pltpu.sync_copy(row_hbm.at[pl.ds(off0, ch)], ri0)
        pltpu.sync_copy(row_hbm.at[pl.ds(off1, ch)], ri1)
        pltpu.sync_copy(col_hbm.at[pl.ds(off0, ch)], ci0)
        pltpu.sync_copy(col_hbm.at[pl.ds(off1, ch)], ci1)
        d0 = pltpu.async_copy(h_hbm.at[ri0], ra0, sem)
        d1 = pltpu.async_copy(h_hbm.at[ri1], ra1, sem)
        d2 = pltpu.async_copy(h_hbm.at[ci0], ca0, sem)
        d3 = pltpu.async_copy(h_hbm.at[ci1], ca1, sem)
        d0.wait(); d1.wait(); d2.wait(); d3.wait()
        w0 = pltpu.async_copy(ra0, hcat_hbm.at[pl.ds(off0, ch), pl.ds(0, d)], wsem)
        w1 = pltpu.async_copy(ra1, hcat_hbm.at[pl.ds(off1, ch), pl.ds(0, d)], wsem)
        w2 = pltpu.async_copy(ca0, hcat_hbm.at[pl.ds(off0, ch), pl.ds(d, d)], wsem)
        w3 = pltpu.async_copy(ca1, hcat_hbm.at[pl.ds(off1, ch), pl.ds(d, d)], wsem)
        w0.wait(); w1.wait(); w2.wait(); w3.wait()
        return 0

    lax.fori_loop(0, epw // (2 * ch), body, 0)


def _sc_gather(h, row, col):
    e = row.shape[0]
    d = h.shape[1]
    epw = e // _NW
    ch = 200
    mesh = plsc.VectorSubcoreMesh(core_axis_name="c", subcore_axis_name="s")
    kern = pl.kernel(
        functools.partial(_gather_body, epw=epw, ch=ch, d=d),
        out_type=jax.ShapeDtypeStruct((e, 2 * d), h.dtype),
        mesh=mesh,
        scratch_types=[
            pltpu.VMEM((ch,), jnp.int32),
            pltpu.VMEM((ch,), jnp.int32),
            pltpu.VMEM((ch,), jnp.int32),
            pltpu.VMEM((ch,), jnp.int32),
            pltpu.VMEM((ch, d), h.dtype),
            pltpu.VMEM((ch, d), h.dtype),
            pltpu.VMEM((ch, d), h.dtype),
            pltpu.VMEM((ch, d), h.dtype),
            pltpu.SemaphoreType.DMA,
            pltpu.SemaphoreType.DMA,
        ],
    )
    return kern(h, row, col)


def _scatter_body(msg_hbm, row_hbm, zeros_hbm, out_hbm,
                  idx_v, msg_v, shared, sem, *, epw, ch, nps):
    c = lax.axis_index("c")
    s = lax.axis_index("s")
    wid = s * _NC + c
    base = wid * epw

    pltpu.sync_copy(zeros_hbm, shared.at[pl.ds(s * nps, nps)])
    plsc.subcore_barrier()

    def body(i, _):
        off = base + i * ch
        d0 = pltpu.async_copy(row_hbm.at[pl.ds(off, ch)], idx_v, sem)
        d1 = pltpu.async_copy(msg_hbm.at[pl.ds(off, ch)], msg_v, sem)
        d0.wait(); d1.wait()
        pltpu.sync_copy(msg_v, shared.at[idx_v], add=True)
        return 0

    lax.fori_loop(0, epw // ch, body, 0)
    plsc.subcore_barrier()

    npad = nps * _NS
    pltpu.sync_copy(shared.at[pl.ds(s * nps, nps)],
                    out_hbm.at[pl.ds(c * npad + s * nps, nps)])


def _sc_scatter(msg, row, n_pad):
    e, d = msg.shape
    epw = e // _NW
    ch = 200
    nps = n_pad // _NS
    zeros = jnp.zeros((nps, d), msg.dtype)
    mesh = plsc.VectorSubcoreMesh(core_axis_name="c", subcore_axis_name="s")
    kern = pl.kernel(
        functools.partial(_scatter_body, epw=epw, ch=ch, nps=nps),
        out_type=jax.ShapeDtypeStruct((_NC * n_pad, d), msg.dtype),
        mesh=mesh,
        scratch_types=[
            pltpu.VMEM((ch,), jnp.int32),
            pltpu.VMEM((ch, d), msg.dtype),
            pltpu.VMEM_SHARED((n_pad, d), msg.dtype),
            pltpu.SemaphoreType.DMA,
        ],
    )
    return kern(msg, row, zeros)


def _mlp_body(x_ref, w1, b1, vw2t, vb2, w2f, w2p, c2, out_ref):
    f32 = jnp.float32
    bf = jnp.bfloat16
    d = x_ref.shape[1] // 2
    x = x_ref[...]
    xb = x.astype(bf)
    z = jnp.dot(xb, w1[...], preferred_element_type=f32) + b1[...]
    tv = jnp.tanh(z[:, :d])
    tf = jax.nn.relu(z[:, d:2 * d])
    tp = jnp.tanh(z[:, 2 * d:])
    s = (jnp.dot(tf.astype(bf), w2f[...], preferred_element_type=f32)
         + jnp.dot(tp.astype(bf), w2p[...], preferred_element_type=f32)
         + c2[...])
    nu = jnp.dot(tv.astype(bf), vw2t[...], preferred_element_type=f32) + vb2[...]
    out_ref[...] = s + nu * (x[:, d:] - x[:, :d])


def _tc_mlp(hcat, weights, d):
    e = hcat.shape[0]
    be = 2000
    grid = e // be
    in_spec = pl.BlockSpec((be, 2 * d), lambda i: (i, 0))
    out_spec = pl.BlockSpec((be, d), lambda i: (i, 0))
    full = lambda a: pl.BlockSpec(a.shape, lambda i: tuple(0 for _ in a.shape))
    return pl.pallas_call(
        _mlp_body,
        out_shape=jax.ShapeDtypeStruct((e, d), jnp.float32),
        grid=(grid,),
        in_specs=[in_spec] + [full(w) for w in weights],
        out_specs=out_spec,
    )(hcat, *weights)


def _combine_body(h_ref, p0_ref, p1_ref, out_ref):
    out_ref[...] = h_ref[...] + DT * (p0_ref[...] + p1_ref[...])


def _tc_combine(h, partials, n_pad):
    n, d = h.shape
    bn = 80
    spec = pl.BlockSpec((bn, d), lambda i: (i, 0))
    p1_spec = pl.BlockSpec((bn, d), lambda i: (i + n_pad // bn, 0))
    return pl.pallas_call(
        _combine_body,
        out_shape=jax.ShapeDtypeStruct((n, d), jnp.float32),
        grid=(n // bn,),
        in_specs=[spec, spec, p1_spec],
        out_specs=spec,
    )(h, partials, partials)


def kernel(h, edge_index, vw1, vb1, vw2, vb2, pw1, pb1, pw2, pb2,
           fw1, fb1, fw2, fb2):
    n, d = h.shape
    f32 = jnp.float32
    bf = jnp.bfloat16
    row = edge_index[0]
    col = edge_index[1]

    hcat = _sc_gather(h, row, col)

    w1 = jnp.concatenate([
        jnp.concatenate([vw1[:d], fw1[:d], pw1], axis=1),
        jnp.concatenate([vw1[d:], fw1[d:], -pw1], axis=1),
    ], axis=0).astype(bf)
    b1 = jnp.concatenate([vb1, fb1, pb1]).reshape(1, 3 * d).astype(f32)
    weights = (
        w1, b1,
        jnp.tile(vw2, (1, d)).astype(bf),
        vb2.reshape(1, 1).astype(f32),
        fw2.astype(bf), (-pw2).astype(bf),
        (fb2 - pb2).reshape(1, d).astype(f32),
    )
    msg = _tc_mlp(hcat, weights, d)

    n_pad = ((n + _NW * 8 - 1) // (_NW * 8)) * (_NW * 8)
    partials = _sc_scatter(msg, row, n_pad)

    return _tc_combine(h, partials, n_pad)


# speedup vs baseline: 3.7871x; 1.1880x over previous
import functools

import jax
import jax.numpy as jnp
from jax import lax
from jax.experimental import pallas as pl
from jax.experimental.pallas import tpu as pltpu
from jax.experimental.pallas import tpu_sc as plsc

DT = 0.03

_NC = 2
_NS = 16
_NW = _NC * _NS


def _gather_body(h_hbm, row_hbm, col_hbm, hcat_hbm,
                 ri0, ri1, ci0, ci1, ra0, ra1, ca0, ca1, sem, wsem,
                 *, epw, ch, d):
    c = lax.axis_index("c")
    s = lax.axis_index("s")
    wid = s * _NC + c
    base = wid * epw

    def body(k, _):
        off0 = base + (2 * k) * ch
        off1 = off0 + ch
        pltpu.sync_copy(row_hbm.at[pl.ds(off0, ch)], ri0)
        pltpu.sync_copy(row_hbm.at[pl.ds(off1, ch)], ri1)
        pltpu.sync_copy(col_hbm.at[pl.ds(off0, ch)], ci0)
        pltpu.sync_copy(col_hbm.at[pl.ds(off1, ch)], ci1)
        d0 = pltpu.async_copy(h_hbm.at[ri0], ra0, sem)
        d1 = pltpu.async_copy(h_hbm.at[ri1], ra1, sem)
        d2 = pltpu.async_copy(h_hbm.at[ci0], ca0, sem)
        d3 = pltpu.async_copy(h_hbm.at[ci1], ca1, sem)
        d0.wait(); d1.wait(); d2.wait(); d3.wait()
        w0 = pltpu.async_copy(ra0, hcat_hbm.at[pl.ds(off0, ch), pl.ds(0, d)], wsem)
        w1 = pltpu.async_copy(ra1, hcat_hbm.at[pl.ds(off1, ch), pl.ds(0, d)], wsem)
        w2 = pltpu.async_copy(ca0, hcat_hbm.at[pl.ds(off0, ch), pl.ds(d, d)], wsem)
        w3 = pltpu.async_copy(ca1, hcat_hbm.at[pl.ds(off1, ch), pl.ds(d, d)], wsem)
        w0.wait(); w1.wait(); w2.wait(); w3.wait()
        return 0

    lax.fori_loop(0, epw // (2 * ch), body, 0)


def _sc_gather(h, row, col):
    e = row.shape[0]
    d = h.shape[1]
    epw = e // _NW
    ch = 200
    mesh = plsc.VectorSubcoreMesh(core_axis_name="c", subcore_axis_name="s")
    kern = pl.kernel(
        functools.partial(_gather_body, epw=epw, ch=ch, d=d),
        out_type=jax.ShapeDtypeStruct((e, 2 * d), h.dtype),
        mesh=mesh,
        scratch_types=[
            pltpu.VMEM((ch,), jnp.int32),
            pltpu.VMEM((ch,), jnp.int32),
            pltpu.VMEM((ch,), jnp.int32),
            pltpu.VMEM((ch,), jnp.int32),
            pltpu.VMEM((ch, d), h.dtype),
            pltpu.VMEM((ch, d), h.dtype),
            pltpu.VMEM((ch, d), h.dtype),
            pltpu.VMEM((ch, d), h.dtype),
            pltpu.SemaphoreType.DMA,
            pltpu.SemaphoreType.DMA,
        ],
    )
    return kern(h, row, col)


def _scatter_body(*refs, epc, ch, nps, nchunks):
    msgs = refs[:nchunks]
    row_hbm, zeros_hbm, out_hbm, idx_v, msg_v, shared, sem = refs[nchunks:]
    c = lax.axis_index("c")
    s = lax.axis_index("s")
    wid = s * _NC + c

    pltpu.sync_copy(zeros_hbm, shared.at[pl.ds(s * nps, nps)])
    plsc.subcore_barrier()

    for q, mref in enumerate(msgs):
        base = wid * epc

        def body(i, _):
            off = base + i * ch
            d0 = pltpu.async_copy(row_hbm.at[pl.ds(q * epc * _NW + off, ch)],
                                  idx_v, sem)
            d1 = pltpu.async_copy(mref.at[pl.ds(off, ch)], msg_v, sem)
            d0.wait(); d1.wait()
            pltpu.sync_copy(msg_v, shared.at[idx_v], add=True)
            return 0

        lax.fori_loop(0, epc // ch, body, 0)

    plsc.subcore_barrier()

    npad = nps * _NS
    pltpu.sync_copy(shared.at[pl.ds(s * nps, nps)],
                    out_hbm.at[pl.ds(c * npad + s * nps, nps)])


def _sc_scatter(msgs, row, n_pad):
    ec, d = msgs[0].shape
    epc = ec // _NW
    ch = 200
    nps = n_pad // _NS
    zeros = jnp.zeros((nps, d), msgs[0].dtype)
    mesh = plsc.VectorSubcoreMesh(core_axis_name="c", subcore_axis_name="s")
    kern = pl.kernel(
        functools.partial(_scatter_body, epc=epc, ch=ch, nps=nps,
                          nchunks=len(msgs)),
        out_type=jax.ShapeDtypeStruct((_NC * n_pad, d), msgs[0].dtype),
        mesh=mesh,
        scratch_types=[
            pltpu.VMEM((ch,), jnp.int32),
            pltpu.VMEM((ch, d), msgs[0].dtype),
            pltpu.VMEM_SHARED((n_pad, d), msgs[0].dtype),
            pltpu.SemaphoreType.DMA,
        ],
    )
    return kern(*msgs, row, zeros)


def _mlp_body(x_ref, w1, b1, vw2t, vb2, w2f, w2p, c2, out_ref):
    f32 = jnp.float32
    bf = jnp.bfloat16
    d = x_ref.shape[1] // 2
    x = x_ref[...]
    xb = x.astype(bf)
    z = jnp.dot(xb, w1[...], preferred_element_type=f32) + b1[...]
    tv = jnp.tanh(z[:, :d])
    tf = jax.nn.relu(z[:, d:2 * d])
    tp = jnp.tanh(z[:, 2 * d:])
    s = (jnp.dot(tf.astype(bf), w2f[...], preferred_element_type=f32)
         + jnp.dot(tp.astype(bf), w2p[...], preferred_element_type=f32)
         + c2[...])
    nu = jnp.dot(tv.astype(bf), vw2t[...], preferred_element_type=f32) + vb2[...]
    out_ref[...] = s + nu * (x[:, d:] - x[:, :d])


def _tc_mlp(hcat, weights, d):
    e = hcat.shape[0]
    be = 2000
    grid = e // be
    in_spec = pl.BlockSpec((be, 2 * d), lambda i: (i, 0))
    out_spec = pl.BlockSpec((be, d), lambda i: (i, 0))
    full = lambda a: pl.BlockSpec(a.shape, lambda i: tuple(0 for _ in a.shape))
    return pl.pallas_call(
        _mlp_body,
        out_shape=jax.ShapeDtypeStruct((e, d), jnp.float32),
        grid=(grid,),
        in_specs=[in_spec] + [full(w) for w in weights],
        out_specs=out_spec,
    )(hcat, *weights)


def _combine_body(h_ref, p0_ref, p1_ref, out_ref):
    out_ref[...] = h_ref[...] + DT * (p0_ref[...] + p1_ref[...])


def _tc_combine(h, partials, n_pad):
    n, d = h.shape
    bn = 80
    spec = pl.BlockSpec((bn, d), lambda i: (i, 0))
    p1_spec = pl.BlockSpec((bn, d), lambda i: (i + n_pad // bn, 0))
    return pl.pallas_call(
        _combine_body,
        out_shape=jax.ShapeDtypeStruct((n, d), jnp.float32),
        grid=(n // bn,),
        in_specs=[spec, spec, p1_spec],
        out_specs=spec,
    )(h, partials, partials)


def kernel(h, edge_index, vw1, vb1, vw2, vb2, pw1, pb1, pw2, pb2,
           fw1, fb1, fw2, fb2):
    n, d = h.shape
    f32 = jnp.float32
    bf = jnp.bfloat16
    row = edge_index[0]
    col = edge_index[1]

    w1 = jnp.concatenate([
        jnp.concatenate([vw1[:d], fw1[:d], pw1], axis=1),
        jnp.concatenate([vw1[d:], fw1[d:], -pw1], axis=1),
    ], axis=0).astype(bf)
    b1 = jnp.concatenate([vb1, fb1, pb1]).reshape(1, 3 * d).astype(f32)
    weights = (
        w1, b1,
        jnp.tile(vw2, (1, d)).astype(bf),
        vb2.reshape(1, 1).astype(f32),
        fw2.astype(bf), (-pw2).astype(bf),
        (fb2 - pb2).reshape(1, d).astype(f32),
    )
    e = row.shape[0]
    nchunks = 5
    ec = e // nchunks
    msgs = []
    for k in range(nchunks):
        sl = slice(k * ec, (k + 1) * ec)
        hcat_k = _sc_gather(h, row[sl], col[sl])
        msgs.append(_tc_mlp(hcat_k, weights, d))

    n_pad = ((n + _NW * 8 - 1) // (_NW * 8)) * (_NW * 8)
    partials = _sc_scatter(msgs, row, n_pad)

    return _tc_combine(h, partials, n_pad)
